# Initial kernel scaffold; baseline (speedup 1.0000x reference)
import functools

import jax
import jax.numpy as jnp
from jax import lax
from jax.experimental import pallas as pl
from jax.experimental.pallas import tpu as pltpu
from jax.experimental.pallas import tpu_sc as plsc

O, T, D, H = 10000, 160000, 128, 128
DOUT = 128

NC, NS = 2, 16
NW = NC * NS
CHUNK = 128
NCHUNKS = T // CHUNK
CPW = -(-NCHUNKS // NW)


def _gather_body(obj_hbm, sidx_hbm, oidx_hbm, sout_hbm, oout_hbm,
                 sidx_v, oidx_v, srows_v, orows_v, sem1, sem2):
    c = lax.axis_index("c")
    s = lax.axis_index("s")
    w = c * NS + s

    def body(j, _):
        cid = w + NW * j

        @pl.when(cid < NCHUNKS)
        def _():
            base = cid * CHUNK
            pltpu.sync_copy(sidx_hbm.at[pl.ds(base, CHUNK)], sidx_v)
            pltpu.sync_copy(oidx_hbm.at[pl.ds(base, CHUNK)], oidx_v)
            cp1 = pltpu.async_copy(obj_hbm.at[sidx_v], srows_v, sem1)
            cp2 = pltpu.async_copy(obj_hbm.at[oidx_v], orows_v, sem2)
            cp1.wait()
            cp2.wait()
            pltpu.sync_copy(srows_v, sout_hbm.at[pl.ds(base, CHUNK)])
            pltpu.sync_copy(orows_v, oout_hbm.at[pl.ds(base, CHUNK)])

        return 0

    lax.fori_loop(0, CPW, body, 0)


_sc_gather = pl.kernel(
    _gather_body,
    out_type=(jax.ShapeDtypeStruct((T, D), jnp.float32),
              jax.ShapeDtypeStruct((T, D), jnp.float32)),
    mesh=plsc.VectorSubcoreMesh(core_axis_name="c", subcore_axis_name="s"),
    scratch_types=[
        pltpu.VMEM((CHUNK,), jnp.int32),
        pltpu.VMEM((CHUNK,), jnp.int32),
        pltpu.VMEM((CHUNK, D), jnp.float32),
        pltpu.VMEM((CHUNK, D), jnp.float32),
        pltpu.SemaphoreType.DMA,
        pltpu.SemaphoreType.DMA,
    ],
)


def _mlp1_body(s_ref, p_ref, o_ref, w1a_ref, b1a_ref, w1b_ref, b1b_ref,
               ns_ref, np_ref, no_ref):
    w1a = w1a_ref[...]
    h = jnp.dot(s_ref[...], w1a[:D, :], preferred_element_type=jnp.float32)
    h += jnp.dot(p_ref[...], w1a[D:2 * D, :], preferred_element_type=jnp.float32)
    h += jnp.dot(o_ref[...], w1a[2 * D:, :], preferred_element_type=jnp.float32)
    h = jnp.maximum(h + b1a_ref[...], 0.0)
    t = jnp.dot(h, w1b_ref[...], preferred_element_type=jnp.float32)
    t = jnp.maximum(t + b1b_ref[...], 0.0)
    ns_ref[...] = t[:, :H]
    np_ref[...] = t[:, H:H + DOUT]
    no_ref[...] = t[:, H + DOUT:]


def _tc_mlp1(sgath, pred, ogath, W1a, b1a, W1b, b1b):
    TM = 2000
    grid = (T // TM,)
    row = lambda i: (i, 0)
    full = lambda i: (0, 0)
    return pl.pallas_call(
        _mlp1_body,
        grid=grid,
        in_specs=[
            pl.BlockSpec((TM, D), row),
            pl.BlockSpec((TM, D), row),
            pl.BlockSpec((TM, D), row),
            pl.BlockSpec((3 * D, H), full),
            pl.BlockSpec((1, H), full),
            pl.BlockSpec((H, 2 * H + DOUT), full),
            pl.BlockSpec((1, 2 * H + DOUT), full),
        ],
        out_specs=[
            pl.BlockSpec((TM, H), row),
            pl.BlockSpec((TM, DOUT), row),
            pl.BlockSpec((TM, H), row),
        ],
        out_shape=[
            jax.ShapeDtypeStruct((T, H), jnp.float32),
            jax.ShapeDtypeStruct((T, DOUT), jnp.float32),
            jax.ShapeDtypeStruct((T, H), jnp.float32),
        ],
    )(sgath, pred, ogath, W1a, b1a.reshape(1, H), W1b,
      b1b.reshape(1, 2 * H + DOUT))


CW = 16


def _scatter_body(news_hbm, newo_hbm, sidx_hbm, oidx_hbm, zerosv_hbm,
                  zerosc_hbm, ones_hbm, pooled_hbm, counts_hbm,
                  acc_sh, cnt_sh, sidx_v, oidx_v, rows_v, ones_v):
    c = lax.axis_index("c")
    s = lax.axis_index("s")
    w = c * NS + s

    @pl.when(s == 0)
    def _():
        pltpu.sync_copy(zerosv_hbm, acc_sh)
        pltpu.sync_copy(zerosc_hbm, cnt_sh)

    pltpu.sync_copy(ones_hbm, ones_v)
    plsc.subcore_barrier()

    def body(j, _):
        cid = w + NW * j

        @pl.when(cid < NCHUNKS)
        def _():
            base = cid * CHUNK
            pltpu.sync_copy(sidx_hbm.at[pl.ds(base, CHUNK)], sidx_v)
            pltpu.sync_copy(oidx_hbm.at[pl.ds(base, CHUNK)], oidx_v)
            pltpu.sync_copy(news_hbm.at[pl.ds(base, CHUNK)], rows_v)
            pltpu.sync_copy(rows_v, acc_sh.at[sidx_v], add=True)
            pltpu.sync_copy(newo_hbm.at[pl.ds(base, CHUNK)], rows_v)
            pltpu.sync_copy(rows_v, acc_sh.at[oidx_v], add=True)
            pltpu.sync_copy(ones_v, cnt_sh.at[sidx_v], add=True)
            pltpu.sync_copy(ones_v, cnt_sh.at[oidx_v], add=True)

        return 0

    lax.fori_loop(0, CPW, body, 0)
    plsc.subcore_barrier()

    rpw = O // NS
    pltpu.sync_copy(acc_sh.at[pl.ds(s * rpw, rpw)],
                    pooled_hbm.at[c, pl.ds(s * rpw, rpw)])
    pltpu.sync_copy(cnt_sh.at[pl.ds(s * rpw, rpw)],
                    counts_hbm.at[c, pl.ds(s * rpw, rpw)])


_sc_scatter = pl.kernel(
    _scatter_body,
    out_type=(jax.ShapeDtypeStruct((NC, O, H), jnp.float32),
              jax.ShapeDtypeStruct((NC, O, CW), jnp.float32)),
    mesh=plsc.VectorSubcoreMesh(core_axis_name="c", subcore_axis_name="s"),
    scratch_types=[
        pltpu.VMEM_SHARED((O, H), jnp.float32),
        pltpu.VMEM_SHARED((O, CW), jnp.float32),
        pltpu.VMEM((CHUNK,), jnp.int32),
        pltpu.VMEM((CHUNK,), jnp.int32),
        pltpu.VMEM((CHUNK, H), jnp.float32),
        pltpu.VMEM((CHUNK, CW), jnp.float32),
    ],
)


def _mlp2_body(p_ref, c_ref, w2a_ref, b2a_ref, w2b_ref, b2b_ref, out_ref):
    pooled = p_ref[0] + p_ref[1]
    cnt = jnp.maximum(c_ref[0, :, :1] + c_ref[1, :, :1], 1.0)
    pooled = pooled / cnt
    h = jnp.dot(pooled, w2a_ref[...], preferred_element_type=jnp.float32)
    h = jnp.maximum(h + b2a_ref[...], 0.0)
    out = jnp.dot(h, w2b_ref[...], preferred_element_type=jnp.float32)
    out_ref[...] = jnp.maximum(out + b2b_ref[...], 0.0)


def _tc_mlp2(pooled, counts, W2a, b2a, W2b, b2b):
    TM = 1000
    grid = (O // TM,)
    return pl.pallas_call(
        _mlp2_body,
        grid=grid,
        in_specs=[
            pl.BlockSpec((NC, TM, H), lambda i: (0, i, 0)),
            pl.BlockSpec((NC, TM, CW), lambda i: (0, i, 0)),
            pl.BlockSpec((H, H), lambda i: (0, 0)),
            pl.BlockSpec((1, H), lambda i: (0, 0)),
            pl.BlockSpec((H, DOUT), lambda i: (0, 0)),
            pl.BlockSpec((1, DOUT), lambda i: (0, 0)),
        ],
        out_specs=pl.BlockSpec((TM, DOUT), lambda i: (i, 0)),
        out_shape=jax.ShapeDtypeStruct((O, DOUT), jnp.float32),
    )(pooled, counts, W2a, b2a.reshape(1, H), W2b, b2b.reshape(1, DOUT))


def kernel(obj_vecs, pred_vecs, edges, W1a, b1a, W1b, b1b, W2a, b2a, W2b, b2b):
    s_idx = edges[:, 0]
    o_idx = edges[:, 1]
    sgath, ogath = _sc_gather(obj_vecs, s_idx, o_idx)
    new_s, new_p, new_o = _tc_mlp1(sgath, pred_vecs, ogath, W1a, b1a, W1b, b1b)
    zerosv = jnp.zeros((O, H), jnp.float32)
    zerosc = jnp.zeros((O, CW), jnp.float32)
    ones = jnp.ones((CHUNK, CW), jnp.float32)
    pooled, counts = _sc_scatter(new_s, new_o, s_idx, o_idx,
                                 zerosv, zerosc, ones)
    new_obj_vecs = _tc_mlp2(pooled, counts, W2a, b2a, W2b, b2b)
    return (new_obj_vecs, new_p)


# speedup vs baseline: 3.2394x; 3.2394x over previous
import jax
import jax.numpy as jnp
from jax import lax
from jax.experimental import pallas as pl
from jax.experimental.pallas import tpu as pltpu
from jax.experimental.pallas import tpu_sc as plsc

O, T, D, H = 10000, 160000, 128, 128
DOUT = 128
OP = 10240

NC, NS = 2, 16
NW = NC * NS
CHUNK = 128
NCHUNKS = T // CHUNK
CPW = -(-NCHUNKS // NW)


def _gather_body(obj_hbm, sidx_hbm, oidx_hbm, sout_hbm, oout_hbm,
                 sidx_v, oidx_v, srows_v, orows_v, sem1, sem2):
    c = lax.axis_index("c")
    s = lax.axis_index("s")
    w = c * NS + s

    def body(j, _):
        cid = w + NW * j

        @pl.when(cid < NCHUNKS)
        def _():
            base = cid * CHUNK
            pltpu.sync_copy(sidx_hbm.at[pl.ds(base, CHUNK)], sidx_v)
            pltpu.sync_copy(oidx_hbm.at[pl.ds(base, CHUNK)], oidx_v)
            cp1 = pltpu.async_copy(obj_hbm.at[sidx_v], srows_v, sem1)
            cp2 = pltpu.async_copy(obj_hbm.at[oidx_v], orows_v, sem2)
            cp1.wait()
            cp2.wait()
            pltpu.sync_copy(srows_v, sout_hbm.at[pl.ds(base, CHUNK)])
            pltpu.sync_copy(orows_v, oout_hbm.at[pl.ds(base, CHUNK)])

        return 0

    lax.fori_loop(0, CPW, body, 0)


_sc_gather = pl.kernel(
    _gather_body,
    out_type=(jax.ShapeDtypeStruct((T, D), jnp.float32),
              jax.ShapeDtypeStruct((T, D), jnp.float32)),
    mesh=plsc.VectorSubcoreMesh(core_axis_name="c", subcore_axis_name="s"),
    scratch_types=[
        pltpu.VMEM((CHUNK,), jnp.int32),
        pltpu.VMEM((CHUNK,), jnp.int32),
        pltpu.VMEM((CHUNK, D), jnp.float32),
        pltpu.VMEM((CHUNK, D), jnp.float32),
        pltpu.SemaphoreType.DMA,
        pltpu.SemaphoreType.DMA,
    ],
)


def _mlp1_body(s_ref, p_ref, o_ref, w1a_ref, b1a_ref, w1b_ref, b1b_ref,
               ns_ref, np_ref, no_ref):
    w1a = w1a_ref[...]
    h = jnp.dot(s_ref[...], w1a[:D, :], preferred_element_type=jnp.float32)
    h += jnp.dot(p_ref[...], w1a[D:2 * D, :], preferred_element_type=jnp.float32)
    h += jnp.dot(o_ref[...], w1a[2 * D:, :], preferred_element_type=jnp.float32)
    h = jnp.maximum(h + b1a_ref[...], 0.0)
    t = jnp.dot(h, w1b_ref[...], preferred_element_type=jnp.float32)
    t = jnp.maximum(t + b1b_ref[...], 0.0)
    ns_ref[...] = t[:, :H]
    np_ref[...] = t[:, H:H + DOUT]
    no_ref[...] = t[:, H + DOUT:]


def _tc_mlp1(sgath, pred, ogath, W1a, b1a, W1b, b1b):
    TM = 2000
    grid = (T // TM,)
    row = lambda i: (i, 0)
    full = lambda i: (0, 0)
    return pl.pallas_call(
        _mlp1_body,
        grid=grid,
        in_specs=[
            pl.BlockSpec((TM, D), row),
            pl.BlockSpec((TM, D), row),
            pl.BlockSpec((TM, D), row),
            pl.BlockSpec((3 * D, H), full),
            pl.BlockSpec((1, H), full),
            pl.BlockSpec((H, 2 * H + DOUT), full),
            pl.BlockSpec((1, 2 * H + DOUT), full),
        ],
        out_specs=[
            pl.BlockSpec((TM, H), row),
            pl.BlockSpec((TM, DOUT), row),
            pl.BlockSpec((TM, H), row),
        ],
        out_shape=[
            jax.ShapeDtypeStruct((T, H), jnp.float32),
            jax.ShapeDtypeStruct((T, DOUT), jnp.float32),
            jax.ShapeDtypeStruct((T, H), jnp.float32),
        ],
    )(sgath, pred, ogath, W1a, b1a.reshape(1, H), W1b,
      b1b.reshape(1, 2 * H + DOUT))


def _scatter_body(news_hbm, newo_hbm, sidx_hbm, oidx_hbm, zeros_hbm,
                  ones_hbm, rid_hbm, pooled_hbm, counts_hbm,
                  acc_sh, sidx_v, oidx_v, rows_v, sem1):
    c = lax.axis_index("c")
    s = lax.axis_index("s")
    w = c * NS + s
    rpw = OP // NS

    pltpu.sync_copy(zeros_hbm, rows_v)
    for k in range(rpw // CHUNK):
        pltpu.sync_copy(rid_hbm.at[pl.ds(s * rpw + k * CHUNK, CHUNK)], sidx_v)
        pltpu.sync_copy(rows_v, acc_sh.at[sidx_v])
    plsc.subcore_barrier()

    def body(j, _):
        cid = w + NW * j

        @pl.when(cid < NCHUNKS)
        def _():
            base = cid * CHUNK
            pltpu.sync_copy(sidx_hbm.at[pl.ds(base, CHUNK)], sidx_v)
            pltpu.sync_copy(oidx_hbm.at[pl.ds(base, CHUNK)], oidx_v)
            pltpu.sync_copy(news_hbm.at[pl.ds(base, CHUNK)], rows_v)
            pltpu.sync_copy(rows_v, acc_sh.at[sidx_v], add=True)
            pltpu.sync_copy(newo_hbm.at[pl.ds(base, CHUNK)], rows_v)
            pltpu.sync_copy(rows_v, acc_sh.at[oidx_v], add=True)

        return 0

    lax.fori_loop(0, CPW, body, 0)
    plsc.subcore_barrier()

    for k in range(rpw // CHUNK):
        off = s * rpw + k * CHUNK
        pltpu.sync_copy(rid_hbm.at[pl.ds(off, CHUNK)], sidx_v)
        pltpu.async_copy(acc_sh.at[sidx_v], rows_v, sem1).wait()
        pltpu.sync_copy(rows_v, pooled_hbm.at[c, pl.ds(off, CHUNK)])
    pltpu.sync_copy(zeros_hbm, rows_v)
    for k in range(rpw // CHUNK):
        pltpu.sync_copy(rid_hbm.at[pl.ds(s * rpw + k * CHUNK, CHUNK)], sidx_v)
        pltpu.sync_copy(rows_v, acc_sh.at[sidx_v])
    pltpu.sync_copy(ones_hbm, rows_v)
    plsc.subcore_barrier()

    def cbody(j, _):
        cid = w + NW * j

        @pl.when(cid < NCHUNKS)
        def _():
            base = cid * CHUNK
            pltpu.sync_copy(sidx_hbm.at[pl.ds(base, CHUNK)], sidx_v)
            pltpu.sync_copy(oidx_hbm.at[pl.ds(base, CHUNK)], oidx_v)
            pltpu.sync_copy(rows_v, acc_sh.at[sidx_v], add=True)
            pltpu.sync_copy(rows_v, acc_sh.at[oidx_v], add=True)

        return 0

    lax.fori_loop(0, CPW, cbody, 0)
    plsc.subcore_barrier()

    for k in range(rpw // CHUNK):
        off = s * rpw + k * CHUNK
        pltpu.sync_copy(rid_hbm.at[pl.ds(off, CHUNK)], sidx_v)
        pltpu.async_copy(acc_sh.at[sidx_v], rows_v, sem1).wait()
        pltpu.sync_copy(rows_v, counts_hbm.at[c, pl.ds(off, CHUNK)])


_sc_scatter = pl.kernel(
    _scatter_body,
    out_type=(jax.ShapeDtypeStruct((NC, OP, H), jnp.float32),
              jax.ShapeDtypeStruct((NC, OP, H), jnp.float32)),
    mesh=plsc.VectorSubcoreMesh(core_axis_name="c", subcore_axis_name="s"),
    scratch_types=[
        pltpu.VMEM_SHARED((OP, H), jnp.float32),
        pltpu.VMEM((CHUNK,), jnp.int32),
        pltpu.VMEM((CHUNK,), jnp.int32),
        pltpu.VMEM((CHUNK, H), jnp.float32),
        pltpu.SemaphoreType.DMA,
    ],
)


def _mlp2_body(p_ref, c_ref, w2a_ref, b2a_ref, w2b_ref, b2b_ref, out_ref):
    pooled = p_ref[0] + p_ref[1]
    cnt = jnp.maximum(c_ref[0, :, :1] + c_ref[1, :, :1], 1.0)
    pooled = pooled / cnt
    h = jnp.dot(pooled, w2a_ref[...], preferred_element_type=jnp.float32)
    h = jnp.maximum(h + b2a_ref[...], 0.0)
    out = jnp.dot(h, w2b_ref[...], preferred_element_type=jnp.float32)
    out_ref[...] = jnp.maximum(out + b2b_ref[...], 0.0)


def _tc_mlp2(pooled, counts, W2a, b2a, W2b, b2b):
    TM = 1024
    grid = (OP // TM,)
    return pl.pallas_call(
        _mlp2_body,
        grid=grid,
        in_specs=[
            pl.BlockSpec((NC, TM, H), lambda i: (0, i, 0)),
            pl.BlockSpec((NC, TM, H), lambda i: (0, i, 0)),
            pl.BlockSpec((H, H), lambda i: (0, 0)),
            pl.BlockSpec((1, H), lambda i: (0, 0)),
            pl.BlockSpec((H, DOUT), lambda i: (0, 0)),
            pl.BlockSpec((1, DOUT), lambda i: (0, 0)),
        ],
        out_specs=pl.BlockSpec((TM, DOUT), lambda i: (i, 0)),
        out_shape=jax.ShapeDtypeStruct((OP, DOUT), jnp.float32),
    )(pooled, counts, W2a, b2a.reshape(1, H), W2b, b2b.reshape(1, DOUT))


def kernel(obj_vecs, pred_vecs, edges, W1a, b1a, W1b, b1b, W2a, b2a, W2b, b2b):
    s_idx = edges[:, 0]
    o_idx = edges[:, 1]
    sgath, ogath = _sc_gather(obj_vecs, s_idx, o_idx)
    new_s, new_p, new_o = _tc_mlp1(sgath, pred_vecs, ogath, W1a, b1a, W1b, b1b)
    zeros = jnp.zeros((CHUNK, H), jnp.float32)
    ones = jnp.ones((CHUNK, H), jnp.float32)
    rid = jnp.arange(OP, dtype=jnp.int32)
    pooled, counts = _sc_scatter(new_s, new_o, s_idx, o_idx, zeros, ones, rid)
    new_obj_vecs = _tc_mlp2(pooled, counts, W2a, b2a, W2b, b2b)
    return (new_obj_vecs[:O], new_p)


# speedup vs baseline: 3.4618x; 1.0687x over previous
import jax
import jax.numpy as jnp
from jax import lax
from jax.experimental import pallas as pl
from jax.experimental.pallas import tpu as pltpu
from jax.experimental.pallas import tpu_sc as plsc

O, T, D, H = 10000, 160000, 128, 128
DOUT = 128
OP = 10240

NC, NS = 2, 16
NW = NC * NS
CHUNK = 128
NCHUNKS = T // CHUNK
CPW = -(-NCHUNKS // NW)


def _gather_body(obj_hbm, sidx_hbm, oidx_hbm, sout_hbm, oout_hbm,
                 sidx0, oidx0, sidx1, oidx1,
                 srows0, orows0, srows1, orows1,
                 s0m, o0m, s1m, o1m):
    c = lax.axis_index("c")
    s = lax.axis_index("s")
    w = c * NS + s
    NPW = NCHUNKS // NW - 1
    start = w * (NCHUNKS // NW)

    def pair(base0):
        base1 = base0 + CHUNK
        pltpu.sync_copy(sidx_hbm.at[pl.ds(base0, CHUNK)], sidx0)
        pltpu.sync_copy(oidx_hbm.at[pl.ds(base0, CHUNK)], oidx0)
        g0s = pltpu.async_copy(obj_hbm.at[sidx0], srows0, s0m)
        g0o = pltpu.async_copy(obj_hbm.at[oidx0], orows0, o0m)
        pltpu.sync_copy(sidx_hbm.at[pl.ds(base1, CHUNK)], sidx1)
        pltpu.sync_copy(oidx_hbm.at[pl.ds(base1, CHUNK)], oidx1)
        g1s = pltpu.async_copy(obj_hbm.at[sidx1], srows1, s1m)
        g1o = pltpu.async_copy(obj_hbm.at[oidx1], orows1, o1m)
        g0s.wait()
        g0o.wait()
        pltpu.sync_copy(srows0, sout_hbm.at[pl.ds(base0, CHUNK)])
        pltpu.sync_copy(orows0, oout_hbm.at[pl.ds(base0, CHUNK)])
        g1s.wait()
        g1o.wait()
        pltpu.sync_copy(srows1, sout_hbm.at[pl.ds(base1, CHUNK)])
        pltpu.sync_copy(orows1, oout_hbm.at[pl.ds(base1, CHUNK)])

    def single(base0):
        pltpu.sync_copy(sidx_hbm.at[pl.ds(base0, CHUNK)], sidx0)
        pltpu.sync_copy(oidx_hbm.at[pl.ds(base0, CHUNK)], oidx0)
        g0s = pltpu.async_copy(obj_hbm.at[sidx0], srows0, s0m)
        g0o = pltpu.async_copy(obj_hbm.at[oidx0], orows0, o0m)
        g0s.wait()
        g0o.wait()
        pltpu.sync_copy(srows0, sout_hbm.at[pl.ds(base0, CHUNK)])
        pltpu.sync_copy(orows0, oout_hbm.at[pl.ds(base0, CHUNK)])

    def body(jj, _):
        pair((start + 2 * jj) * CHUNK)
        return 0

    lax.fori_loop(0, NPW // 2, body, 0)
    single((start + NPW) * CHUNK)

    @pl.when(w < NCHUNKS - NW * (NCHUNKS // NW))
    def _():
        single((NW * (NCHUNKS // NW) + w) * CHUNK)


_sc_gather = pl.kernel(
    _gather_body,
    out_type=(jax.ShapeDtypeStruct((T, D), jnp.float32),
              jax.ShapeDtypeStruct((T, D), jnp.float32)),
    mesh=plsc.VectorSubcoreMesh(core_axis_name="c", subcore_axis_name="s"),
    scratch_types=[
        pltpu.VMEM((CHUNK,), jnp.int32),
        pltpu.VMEM((CHUNK,), jnp.int32),
        pltpu.VMEM((CHUNK,), jnp.int32),
        pltpu.VMEM((CHUNK,), jnp.int32),
        pltpu.VMEM((CHUNK, D), jnp.float32),
        pltpu.VMEM((CHUNK, D), jnp.float32),
        pltpu.VMEM((CHUNK, D), jnp.float32),
        pltpu.VMEM((CHUNK, D), jnp.float32),
        pltpu.SemaphoreType.DMA,
        pltpu.SemaphoreType.DMA,
        pltpu.SemaphoreType.DMA,
        pltpu.SemaphoreType.DMA,
    ],
)


def _mlp1_body(s_ref, p_ref, o_ref, w1a_ref, b1a_ref, w1b_ref, b1b_ref,
               ns_ref, np_ref, no_ref):
    bf = jnp.bfloat16
    w1a = w1a_ref[...].astype(bf)
    h = jnp.dot(s_ref[...].astype(bf), w1a[:D, :],
                preferred_element_type=jnp.float32)
    h += jnp.dot(p_ref[...].astype(bf), w1a[D:2 * D, :],
                 preferred_element_type=jnp.float32)
    h += jnp.dot(o_ref[...].astype(bf), w1a[2 * D:, :],
                 preferred_element_type=jnp.float32)
    h = jnp.maximum(h + b1a_ref[...], 0.0)
    t = jnp.dot(h.astype(bf), w1b_ref[...].astype(bf),
                preferred_element_type=jnp.float32)
    t = jnp.maximum(t + b1b_ref[...], 0.0)
    ns_ref[...] = t[:, :H]
    np_ref[...] = t[:, H:H + DOUT]
    no_ref[...] = t[:, H + DOUT:]


def _tc_mlp1(sgath, pred, ogath, W1a, b1a, W1b, b1b):
    TM = 2000
    grid = (T // TM,)
    row = lambda i: (i, 0)
    full = lambda i: (0, 0)
    return pl.pallas_call(
        _mlp1_body,
        grid=grid,
        in_specs=[
            pl.BlockSpec((TM, D), row),
            pl.BlockSpec((TM, D), row),
            pl.BlockSpec((TM, D), row),
            pl.BlockSpec((3 * D, H), full),
            pl.BlockSpec((1, H), full),
            pl.BlockSpec((H, 2 * H + DOUT), full),
            pl.BlockSpec((1, 2 * H + DOUT), full),
        ],
        out_specs=[
            pl.BlockSpec((TM, H), row),
            pl.BlockSpec((TM, DOUT), row),
            pl.BlockSpec((TM, H), row),
        ],
        out_shape=[
            jax.ShapeDtypeStruct((T, H), jnp.float32),
            jax.ShapeDtypeStruct((T, DOUT), jnp.float32),
            jax.ShapeDtypeStruct((T, H), jnp.float32),
        ],
    )(sgath, pred, ogath, W1a, b1a.reshape(1, H), W1b,
      b1b.reshape(1, 2 * H + DOUT))


def _scatter_body(news_hbm, newo_hbm, sidx_hbm, oidx_hbm, zeros_hbm,
                  ones_hbm, rid_hbm, pooled_hbm, counts_hbm,
                  acc_sh, sidx_v, oidx_v, rows_v, sem1):
    c = lax.axis_index("c")
    s = lax.axis_index("s")
    w = c * NS + s
    rpw = OP // NS

    pltpu.sync_copy(zeros_hbm, rows_v)
    for k in range(rpw // CHUNK):
        pltpu.sync_copy(rid_hbm.at[pl.ds(s * rpw + k * CHUNK, CHUNK)], sidx_v)
        pltpu.sync_copy(rows_v, acc_sh.at[sidx_v])
    plsc.subcore_barrier()

    def body(j, _):
        cid = w + NW * j

        @pl.when(cid < NCHUNKS)
        def _():
            base = cid * CHUNK
            pltpu.sync_copy(sidx_hbm.at[pl.ds(base, CHUNK)], sidx_v)
            pltpu.sync_copy(oidx_hbm.at[pl.ds(base, CHUNK)], oidx_v)
            pltpu.sync_copy(news_hbm.at[pl.ds(base, CHUNK)], rows_v)
            pltpu.sync_copy(rows_v, acc_sh.at[sidx_v], add=True)
            pltpu.sync_copy(newo_hbm.at[pl.ds(base, CHUNK)], rows_v)
            pltpu.sync_copy(rows_v, acc_sh.at[oidx_v], add=True)

        return 0

    lax.fori_loop(0, CPW, body, 0)
    plsc.subcore_barrier()

    for k in range(rpw // CHUNK):
        off = s * rpw + k * CHUNK
        pltpu.sync_copy(rid_hbm.at[pl.ds(off, CHUNK)], sidx_v)
        pltpu.async_copy(acc_sh.at[sidx_v], rows_v, sem1).wait()
        pltpu.sync_copy(rows_v, pooled_hbm.at[c, pl.ds(off, CHUNK)])
    pltpu.sync_copy(zeros_hbm, rows_v)
    for k in range(rpw // CHUNK):
        pltpu.sync_copy(rid_hbm.at[pl.ds(s * rpw + k * CHUNK, CHUNK)], sidx_v)
        pltpu.sync_copy(rows_v, acc_sh.at[sidx_v])
    pltpu.sync_copy(ones_hbm, rows_v)
    plsc.subcore_barrier()

    def cbody(j, _):
        cid = w + NW * j

        @pl.when(cid < NCHUNKS)
        def _():
            base = cid * CHUNK
            pltpu.sync_copy(sidx_hbm.at[pl.ds(base, CHUNK)], sidx_v)
            pltpu.sync_copy(oidx_hbm.at[pl.ds(base, CHUNK)], oidx_v)
            pltpu.sync_copy(rows_v, acc_sh.at[sidx_v], add=True)
            pltpu.sync_copy(rows_v, acc_sh.at[oidx_v], add=True)

        return 0

    lax.fori_loop(0, CPW, cbody, 0)
    plsc.subcore_barrier()

    for k in range(rpw // CHUNK):
        off = s * rpw + k * CHUNK
        pltpu.sync_copy(rid_hbm.at[pl.ds(off, CHUNK)], sidx_v)
        pltpu.async_copy(acc_sh.at[sidx_v], rows_v, sem1).wait()
        pltpu.sync_copy(rows_v, counts_hbm.at[c, pl.ds(off, CHUNK)])


_sc_scatter = pl.kernel(
    _scatter_body,
    out_type=(jax.ShapeDtypeStruct((NC, OP, H), jnp.float32),
              jax.ShapeDtypeStruct((NC, OP, H), jnp.float32)),
    mesh=plsc.VectorSubcoreMesh(core_axis_name="c", subcore_axis_name="s"),
    scratch_types=[
        pltpu.VMEM_SHARED((OP, H), jnp.float32),
        pltpu.VMEM((CHUNK,), jnp.int32),
        pltpu.VMEM((CHUNK,), jnp.int32),
        pltpu.VMEM((CHUNK, H), jnp.float32),
        pltpu.SemaphoreType.DMA,
    ],
)


def _mlp2_body(p_ref, c_ref, w2a_ref, b2a_ref, w2b_ref, b2b_ref, out_ref):
    pooled = p_ref[0] + p_ref[1]
    cnt = jnp.maximum(c_ref[0, :, :1] + c_ref[1, :, :1], 1.0)
    pooled = pooled / cnt
    h = jnp.dot(pooled, w2a_ref[...], preferred_element_type=jnp.float32)
    h = jnp.maximum(h + b2a_ref[...], 0.0)
    out = jnp.dot(h, w2b_ref[...], preferred_element_type=jnp.float32)
    out_ref[...] = jnp.maximum(out + b2b_ref[...], 0.0)


def _tc_mlp2(pooled, counts, W2a, b2a, W2b, b2b):
    TM = 1024
    grid = (OP // TM,)
    return pl.pallas_call(
        _mlp2_body,
        grid=grid,
        in_specs=[
            pl.BlockSpec((NC, TM, H), lambda i: (0, i, 0)),
            pl.BlockSpec((NC, TM, H), lambda i: (0, i, 0)),
            pl.BlockSpec((H, H), lambda i: (0, 0)),
            pl.BlockSpec((1, H), lambda i: (0, 0)),
            pl.BlockSpec((H, DOUT), lambda i: (0, 0)),
            pl.BlockSpec((1, DOUT), lambda i: (0, 0)),
        ],
        out_specs=pl.BlockSpec((TM, DOUT), lambda i: (i, 0)),
        out_shape=jax.ShapeDtypeStruct((OP, DOUT), jnp.float32),
    )(pooled, counts, W2a, b2a.reshape(1, H), W2b, b2b.reshape(1, DOUT))


def kernel(obj_vecs, pred_vecs, edges, W1a, b1a, W1b, b1b, W2a, b2a, W2b, b2b):
    s_idx = edges[:, 0]
    o_idx = edges[:, 1]
    sgath, ogath = _sc_gather(obj_vecs, s_idx, o_idx)
    new_s, new_p, new_o = _tc_mlp1(sgath, pred_vecs, ogath, W1a, b1a, W1b, b1b)
    zeros = jnp.zeros((CHUNK, H), jnp.float32)
    ones = jnp.ones((CHUNK, H), jnp.float32)
    rid = jnp.arange(OP, dtype=jnp.int32)
    pooled, counts = _sc_scatter(new_s, new_o, s_idx, o_idx, zeros, ones, rid)
    new_obj_vecs = _tc_mlp2(pooled, counts, W2a, b2a, W2b, b2b)
    return (new_obj_vecs[:O], new_p)


# speedup vs baseline: 3.5901x; 1.0370x over previous
import jax
import jax.numpy as jnp
from jax import lax
from jax.experimental import pallas as pl
from jax.experimental.pallas import tpu as pltpu
from jax.experimental.pallas import tpu_sc as plsc

O, T, D, H = 10000, 160000, 128, 128
DOUT = 128
OP = 10240

NC, NS = 2, 16
NW = NC * NS
CHUNK = 128
NCHUNKS = T // CHUNK
CPW = -(-NCHUNKS // NW)


def _gather_body(obj_hbm, sidx_hbm, oidx_hbm, sout_hbm, oout_hbm,
                 sidx0, oidx0, sidx1, oidx1,
                 srows0, orows0, srows1, orows1,
                 s0m, o0m, s1m, o1m):
    c = lax.axis_index("c")
    s = lax.axis_index("s")
    w = c * NS + s
    NPW = NCHUNKS // NW - 1
    start = w * (NCHUNKS // NW)

    def pair(base0):
        base1 = base0 + CHUNK
        pltpu.sync_copy(sidx_hbm.at[pl.ds(base0, CHUNK)], sidx0)
        pltpu.sync_copy(oidx_hbm.at[pl.ds(base0, CHUNK)], oidx0)
        g0s = pltpu.async_copy(obj_hbm.at[sidx0], srows0, s0m)
        g0o = pltpu.async_copy(obj_hbm.at[oidx0], orows0, o0m)
        pltpu.sync_copy(sidx_hbm.at[pl.ds(base1, CHUNK)], sidx1)
        pltpu.sync_copy(oidx_hbm.at[pl.ds(base1, CHUNK)], oidx1)
        g1s = pltpu.async_copy(obj_hbm.at[sidx1], srows1, s1m)
        g1o = pltpu.async_copy(obj_hbm.at[oidx1], orows1, o1m)
        g0s.wait()
        g0o.wait()
        pltpu.sync_copy(srows0, sout_hbm.at[pl.ds(base0, CHUNK)])
        pltpu.sync_copy(orows0, oout_hbm.at[pl.ds(base0, CHUNK)])
        g1s.wait()
        g1o.wait()
        pltpu.sync_copy(srows1, sout_hbm.at[pl.ds(base1, CHUNK)])
        pltpu.sync_copy(orows1, oout_hbm.at[pl.ds(base1, CHUNK)])

    def single(base0):
        pltpu.sync_copy(sidx_hbm.at[pl.ds(base0, CHUNK)], sidx0)
        pltpu.sync_copy(oidx_hbm.at[pl.ds(base0, CHUNK)], oidx0)
        g0s = pltpu.async_copy(obj_hbm.at[sidx0], srows0, s0m)
        g0o = pltpu.async_copy(obj_hbm.at[oidx0], orows0, o0m)
        g0s.wait()
        g0o.wait()
        pltpu.sync_copy(srows0, sout_hbm.at[pl.ds(base0, CHUNK)])
        pltpu.sync_copy(orows0, oout_hbm.at[pl.ds(base0, CHUNK)])

    def body(jj, _):
        pair((start + 2 * jj) * CHUNK)
        return 0

    lax.fori_loop(0, NPW // 2, body, 0)
    single((start + NPW) * CHUNK)

    @pl.when(w < NCHUNKS - NW * (NCHUNKS // NW))
    def _():
        single((NW * (NCHUNKS // NW) + w) * CHUNK)


_sc_gather = pl.kernel(
    _gather_body,
    out_type=(jax.ShapeDtypeStruct((T, D), jnp.float32),
              jax.ShapeDtypeStruct((T, D), jnp.float32)),
    mesh=plsc.VectorSubcoreMesh(core_axis_name="c", subcore_axis_name="s"),
    scratch_types=[
        pltpu.VMEM((CHUNK,), jnp.int32),
        pltpu.VMEM((CHUNK,), jnp.int32),
        pltpu.VMEM((CHUNK,), jnp.int32),
        pltpu.VMEM((CHUNK,), jnp.int32),
        pltpu.VMEM((CHUNK, D), jnp.float32),
        pltpu.VMEM((CHUNK, D), jnp.float32),
        pltpu.VMEM((CHUNK, D), jnp.float32),
        pltpu.VMEM((CHUNK, D), jnp.float32),
        pltpu.SemaphoreType.DMA,
        pltpu.SemaphoreType.DMA,
        pltpu.SemaphoreType.DMA,
        pltpu.SemaphoreType.DMA,
    ],
)


def _mlp1_body(s_ref, p_ref, o_ref, w1a_ref, b1a_ref, w1b_ref, b1b_ref,
               ns_ref, np_ref, no_ref):
    bf = jnp.bfloat16
    w1a = w1a_ref[...].astype(bf)
    h = jnp.dot(s_ref[...].astype(bf), w1a[:D, :],
                preferred_element_type=jnp.float32)
    h += jnp.dot(p_ref[...].astype(bf), w1a[D:2 * D, :],
                 preferred_element_type=jnp.float32)
    h += jnp.dot(o_ref[...].astype(bf), w1a[2 * D:, :],
                 preferred_element_type=jnp.float32)
    h = jnp.maximum(h + b1a_ref[...], 0.0)
    t = jnp.dot(h.astype(bf), w1b_ref[...].astype(bf),
                preferred_element_type=jnp.float32)
    t = jnp.maximum(t + b1b_ref[...], 0.0)
    ns_ref[...] = t[:, :H]
    np_ref[...] = t[:, H:H + DOUT]
    no_ref[...] = t[:, H + DOUT:]


def _tc_mlp1(sgath, pred, ogath, W1a, b1a, W1b, b1b):
    TM = 2000
    grid = (T // TM,)
    row = lambda i: (i, 0)
    full = lambda i: (0, 0)
    return pl.pallas_call(
        _mlp1_body,
        grid=grid,
        in_specs=[
            pl.BlockSpec((TM, D), row),
            pl.BlockSpec((TM, D), row),
            pl.BlockSpec((TM, D), row),
            pl.BlockSpec((3 * D, H), full),
            pl.BlockSpec((1, H), full),
            pl.BlockSpec((H, 2 * H + DOUT), full),
            pl.BlockSpec((1, 2 * H + DOUT), full),
        ],
        out_specs=[
            pl.BlockSpec((TM, H), row),
            pl.BlockSpec((TM, DOUT), row),
            pl.BlockSpec((TM, H), row),
        ],
        out_shape=[
            jax.ShapeDtypeStruct((T, H), jnp.float32),
            jax.ShapeDtypeStruct((T, DOUT), jnp.float32),
            jax.ShapeDtypeStruct((T, H), jnp.float32),
        ],
    )(sgath, pred, ogath, W1a, b1a.reshape(1, H), W1b,
      b1b.reshape(1, 2 * H + DOUT))


def _scatter_body(news_hbm, newo_hbm, sidx_hbm, oidx_hbm, zeros_hbm,
                  ones_hbm, rid_hbm, pooled_hbm, counts_hbm,
                  acc_sh, sidx0, oidx0, sidx1, oidx1, rows0, rows1,
                  sem1, sem2):
    c = lax.axis_index("c")
    s = lax.axis_index("s")
    w = c * NS + s
    rpw = OP // NS
    CPW0 = NCHUNKS // NW
    start = w * CPW0

    pltpu.sync_copy(zeros_hbm, rows0)
    for k in range(rpw // CHUNK):
        pltpu.sync_copy(rid_hbm.at[pl.ds(s * rpw + k * CHUNK, CHUNK)], sidx0)
        pltpu.sync_copy(rows0, acc_sh.at[sidx0])
    plsc.subcore_barrier()

    def add_chunk(base):
        pltpu.sync_copy(sidx_hbm.at[pl.ds(base, CHUNK)], sidx0)
        pltpu.sync_copy(oidx_hbm.at[pl.ds(base, CHUNK)], oidx0)
        ls = pltpu.async_copy(news_hbm.at[pl.ds(base, CHUNK)], rows0, sem1)
        lo = pltpu.async_copy(newo_hbm.at[pl.ds(base, CHUNK)], rows1, sem2)
        ls.wait()
        pltpu.sync_copy(rows0, acc_sh.at[sidx0], add=True)
        lo.wait()
        pltpu.sync_copy(rows1, acc_sh.at[oidx0], add=True)

    def body(j, _):
        add_chunk((start + j) * CHUNK)
        return 0

    lax.fori_loop(0, CPW0, body, 0)

    @pl.when(w < NCHUNKS - NW * CPW0)
    def _():
        add_chunk((NW * CPW0 + w) * CHUNK)

    plsc.subcore_barrier()

    for k in range(rpw // CHUNK):
        off = s * rpw + k * CHUNK
        pltpu.sync_copy(rid_hbm.at[pl.ds(off, CHUNK)], sidx0)
        pltpu.async_copy(acc_sh.at[sidx0], rows0, sem1).wait()
        pltpu.sync_copy(rows0, pooled_hbm.at[c, pl.ds(off, CHUNK)])
    pltpu.sync_copy(zeros_hbm, rows0)
    for k in range(rpw // CHUNK):
        pltpu.sync_copy(rid_hbm.at[pl.ds(s * rpw + k * CHUNK, CHUNK)], sidx0)
        pltpu.sync_copy(rows0, acc_sh.at[sidx0])
    pltpu.sync_copy(ones_hbm, rows0)
    plsc.subcore_barrier()

    def cnt_chunk(base):
        pltpu.sync_copy(sidx_hbm.at[pl.ds(base, CHUNK)], sidx0)
        pltpu.sync_copy(oidx_hbm.at[pl.ds(base, CHUNK)], oidx0)
        pltpu.sync_copy(rows0, acc_sh.at[sidx0], add=True)
        pltpu.sync_copy(rows0, acc_sh.at[oidx0], add=True)

    def cbody(j, _):
        cnt_chunk((start + j) * CHUNK)
        return 0

    lax.fori_loop(0, CPW0, cbody, 0)

    @pl.when(w < NCHUNKS - NW * CPW0)
    def _():
        cnt_chunk((NW * CPW0 + w) * CHUNK)

    plsc.subcore_barrier()

    for k in range(rpw // CHUNK):
        off = s * rpw + k * CHUNK
        pltpu.sync_copy(rid_hbm.at[pl.ds(off, CHUNK)], sidx0)
        pltpu.async_copy(acc_sh.at[sidx0], rows0, sem1).wait()
        pltpu.sync_copy(rows0, counts_hbm.at[c, pl.ds(off, CHUNK)])


_sc_scatter = pl.kernel(
    _scatter_body,
    out_type=(jax.ShapeDtypeStruct((NC, OP, H), jnp.float32),
              jax.ShapeDtypeStruct((NC, OP, H), jnp.float32)),
    mesh=plsc.VectorSubcoreMesh(core_axis_name="c", subcore_axis_name="s"),
    scratch_types=[
        pltpu.VMEM_SHARED((OP, H), jnp.float32),
        pltpu.VMEM((CHUNK,), jnp.int32),
        pltpu.VMEM((CHUNK,), jnp.int32),
        pltpu.VMEM((CHUNK,), jnp.int32),
        pltpu.VMEM((CHUNK,), jnp.int32),
        pltpu.VMEM((CHUNK, H), jnp.float32),
        pltpu.VMEM((CHUNK, H), jnp.float32),
        pltpu.SemaphoreType.DMA,
        pltpu.SemaphoreType.DMA,
    ],
)


def _mlp2_body(p_ref, c_ref, w2a_ref, b2a_ref, w2b_ref, b2b_ref, out_ref):
    pooled = p_ref[0] + p_ref[1]
    cnt = jnp.maximum(c_ref[0, :, :1] + c_ref[1, :, :1], 1.0)
    pooled = pooled / cnt
    h = jnp.dot(pooled, w2a_ref[...], preferred_element_type=jnp.float32)
    h = jnp.maximum(h + b2a_ref[...], 0.0)
    out = jnp.dot(h, w2b_ref[...], preferred_element_type=jnp.float32)
    out_ref[...] = jnp.maximum(out + b2b_ref[...], 0.0)


def _tc_mlp2(pooled, counts, W2a, b2a, W2b, b2b):
    TM = 1024
    grid = (OP // TM,)
    return pl.pallas_call(
        _mlp2_body,
        grid=grid,
        in_specs=[
            pl.BlockSpec((NC, TM, H), lambda i: (0, i, 0)),
            pl.BlockSpec((NC, TM, H), lambda i: (0, i, 0)),
            pl.BlockSpec((H, H), lambda i: (0, 0)),
            pl.BlockSpec((1, H), lambda i: (0, 0)),
            pl.BlockSpec((H, DOUT), lambda i: (0, 0)),
            pl.BlockSpec((1, DOUT), lambda i: (0, 0)),
        ],
        out_specs=pl.BlockSpec((TM, DOUT), lambda i: (i, 0)),
        out_shape=jax.ShapeDtypeStruct((OP, DOUT), jnp.float32),
    )(pooled, counts, W2a, b2a.reshape(1, H), W2b, b2b.reshape(1, DOUT))


def kernel(obj_vecs, pred_vecs, edges, W1a, b1a, W1b, b1b, W2a, b2a, W2b, b2b):
    s_idx = edges[:, 0]
    o_idx = edges[:, 1]
    sgath, ogath = _sc_gather(obj_vecs, s_idx, o_idx)
    new_s, new_p, new_o = _tc_mlp1(sgath, pred_vecs, ogath, W1a, b1a, W1b, b1b)
    zeros = jnp.zeros((CHUNK, H), jnp.float32)
    ones = jnp.ones((CHUNK, H), jnp.float32)
    rid = jnp.arange(OP, dtype=jnp.int32)
    pooled, counts = _sc_scatter(new_s, new_o, s_idx, o_idx, zeros, ones, rid)
    new_obj_vecs = _tc_mlp2(pooled, counts, W2a, b2a, W2b, b2b)
    return (new_obj_vecs[:O], new_p)


# speedup vs baseline: 3.7262x; 1.0379x over previous
import jax
import jax.numpy as jnp
from jax import lax
from jax.experimental import pallas as pl
from jax.experimental.pallas import tpu as pltpu
from jax.experimental.pallas import tpu_sc as plsc

O, T, D, H = 10000, 160000, 128, 128
DOUT = 128
OP = 10240

NC, NS = 2, 16
NW = NC * NS
CHUNK = 128
NCHUNKS = T // CHUNK
CPW = -(-NCHUNKS // NW)


def _gather_body(obj_hbm, sidx_hbm, oidx_hbm, sout_hbm, oout_hbm,
                 sidx0, oidx0, sidx1, oidx1,
                 srows0, orows0, srows1, orows1,
                 s0m, o0m, s1m, o1m):
    c = lax.axis_index("c")
    s = lax.axis_index("s")
    w = c * NS + s
    NPW = NCHUNKS // NW - 1
    start = w * (NCHUNKS // NW)

    def pair(base0):
        base1 = base0 + CHUNK
        pltpu.sync_copy(sidx_hbm.at[pl.ds(base0, CHUNK)], sidx0)
        pltpu.sync_copy(oidx_hbm.at[pl.ds(base0, CHUNK)], oidx0)
        g0s = pltpu.async_copy(obj_hbm.at[sidx0], srows0, s0m)
        g0o = pltpu.async_copy(obj_hbm.at[oidx0], orows0, o0m)
        pltpu.sync_copy(sidx_hbm.at[pl.ds(base1, CHUNK)], sidx1)
        pltpu.sync_copy(oidx_hbm.at[pl.ds(base1, CHUNK)], oidx1)
        g1s = pltpu.async_copy(obj_hbm.at[sidx1], srows1, s1m)
        g1o = pltpu.async_copy(obj_hbm.at[oidx1], orows1, o1m)
        g0s.wait()
        g0o.wait()
        pltpu.sync_copy(srows0, sout_hbm.at[pl.ds(base0, CHUNK)])
        pltpu.sync_copy(orows0, oout_hbm.at[pl.ds(base0, CHUNK)])
        g1s.wait()
        g1o.wait()
        pltpu.sync_copy(srows1, sout_hbm.at[pl.ds(base1, CHUNK)])
        pltpu.sync_copy(orows1, oout_hbm.at[pl.ds(base1, CHUNK)])

    def single(base0):
        pltpu.sync_copy(sidx_hbm.at[pl.ds(base0, CHUNK)], sidx0)
        pltpu.sync_copy(oidx_hbm.at[pl.ds(base0, CHUNK)], oidx0)
        g0s = pltpu.async_copy(obj_hbm.at[sidx0], srows0, s0m)
        g0o = pltpu.async_copy(obj_hbm.at[oidx0], orows0, o0m)
        g0s.wait()
        g0o.wait()
        pltpu.sync_copy(srows0, sout_hbm.at[pl.ds(base0, CHUNK)])
        pltpu.sync_copy(orows0, oout_hbm.at[pl.ds(base0, CHUNK)])

    def body(jj, _):
        pair((start + 2 * jj) * CHUNK)
        return 0

    lax.fori_loop(0, NPW // 2, body, 0)
    single((start + NPW) * CHUNK)

    @pl.when(w < NCHUNKS - NW * (NCHUNKS // NW))
    def _():
        single((NW * (NCHUNKS // NW) + w) * CHUNK)


_sc_gather = pl.kernel(
    _gather_body,
    out_type=(jax.ShapeDtypeStruct((T, D), jnp.float32),
              jax.ShapeDtypeStruct((T, D), jnp.float32)),
    mesh=plsc.VectorSubcoreMesh(core_axis_name="c", subcore_axis_name="s"),
    scratch_types=[
        pltpu.VMEM((CHUNK,), jnp.int32),
        pltpu.VMEM((CHUNK,), jnp.int32),
        pltpu.VMEM((CHUNK,), jnp.int32),
        pltpu.VMEM((CHUNK,), jnp.int32),
        pltpu.VMEM((CHUNK, D), jnp.float32),
        pltpu.VMEM((CHUNK, D), jnp.float32),
        pltpu.VMEM((CHUNK, D), jnp.float32),
        pltpu.VMEM((CHUNK, D), jnp.float32),
        pltpu.SemaphoreType.DMA,
        pltpu.SemaphoreType.DMA,
        pltpu.SemaphoreType.DMA,
        pltpu.SemaphoreType.DMA,
    ],
)


def _mlp1_body(s_ref, p_ref, o_ref, w1a_ref, b1a_ref, w1b_ref, b1b_ref,
               ns_ref, np_ref, no_ref):
    bf = jnp.bfloat16
    w1a = w1a_ref[...].astype(bf)
    h = jnp.dot(s_ref[...].astype(bf), w1a[:D, :],
                preferred_element_type=jnp.float32)
    h += jnp.dot(p_ref[...].astype(bf), w1a[D:2 * D, :],
                 preferred_element_type=jnp.float32)
    h += jnp.dot(o_ref[...].astype(bf), w1a[2 * D:, :],
                 preferred_element_type=jnp.float32)
    h = jnp.maximum(h + b1a_ref[...], 0.0)
    t = jnp.dot(h.astype(bf), w1b_ref[...].astype(bf),
                preferred_element_type=jnp.float32)
    t = jnp.maximum(t + b1b_ref[...], 0.0)
    ns_ref[...] = t[:, :H]
    np_ref[...] = t[:, H:H + DOUT]
    no_ref[...] = t[:, H + DOUT:]


def _tc_mlp1(sgath, pred, ogath, W1a, b1a, W1b, b1b):
    TM = 4000
    grid = (T // TM,)
    row = lambda i: (i, 0)
    full = lambda i: (0, 0)
    return pl.pallas_call(
        _mlp1_body,
        grid=grid,
        in_specs=[
            pl.BlockSpec((TM, D), row),
            pl.BlockSpec((TM, D), row),
            pl.BlockSpec((TM, D), row),
            pl.BlockSpec((3 * D, H), full),
            pl.BlockSpec((1, H), full),
            pl.BlockSpec((H, 2 * H + DOUT), full),
            pl.BlockSpec((1, 2 * H + DOUT), full),
        ],
        out_specs=[
            pl.BlockSpec((TM, H), row),
            pl.BlockSpec((TM, DOUT), row),
            pl.BlockSpec((TM, H), row),
        ],
        out_shape=[
            jax.ShapeDtypeStruct((T, H), jnp.float32),
            jax.ShapeDtypeStruct((T, DOUT), jnp.float32),
            jax.ShapeDtypeStruct((T, H), jnp.float32),
        ],
    )(sgath, pred, ogath, W1a, b1a.reshape(1, H), W1b,
      b1b.reshape(1, 2 * H + DOUT))


def _scatter_body(news_hbm, newo_hbm, sidx_hbm, oidx_hbm, zeros_hbm,
                  ones_hbm, rid_hbm, pooled_hbm, counts_hbm,
                  acc_sh, sidx0, oidx0, sidx1, oidx1, rows0, rows1,
                  sem1, sem2):
    c = lax.axis_index("c")
    s = lax.axis_index("s")
    w = c * NS + s
    rpw = OP // NS
    CPW0 = NCHUNKS // NW
    start = w * CPW0

    pltpu.sync_copy(zeros_hbm, rows0)
    for k in range(rpw // CHUNK):
        pltpu.sync_copy(rid_hbm.at[pl.ds(s * rpw + k * CHUNK, CHUNK)], sidx0)
        pltpu.sync_copy(rows0, acc_sh.at[sidx0])
    plsc.subcore_barrier()

    def add_chunk(base):
        pltpu.sync_copy(sidx_hbm.at[pl.ds(base, CHUNK)], sidx0)
        pltpu.sync_copy(oidx_hbm.at[pl.ds(base, CHUNK)], oidx0)
        ls = pltpu.async_copy(news_hbm.at[pl.ds(base, CHUNK)], rows0, sem1)
        lo = pltpu.async_copy(newo_hbm.at[pl.ds(base, CHUNK)], rows1, sem2)
        ls.wait()
        pltpu.sync_copy(rows0, acc_sh.at[sidx0], add=True)
        lo.wait()
        pltpu.sync_copy(rows1, acc_sh.at[oidx0], add=True)

    def body(j, _):
        add_chunk((start + j) * CHUNK)
        return 0

    lax.fori_loop(0, CPW0, body, 0)

    @pl.when(w < NCHUNKS - NW * CPW0)
    def _():
        add_chunk((NW * CPW0 + w) * CHUNK)

    plsc.subcore_barrier()

    for k in range(rpw // CHUNK):
        off = s * rpw + k * CHUNK
        pltpu.sync_copy(rid_hbm.at[pl.ds(off, CHUNK)], sidx0)
        pltpu.async_copy(acc_sh.at[sidx0], rows0, sem1).wait()
        pltpu.sync_copy(rows0, pooled_hbm.at[c, pl.ds(off, CHUNK)])
    pltpu.sync_copy(zeros_hbm, rows0)
    for k in range(rpw // CHUNK):
        pltpu.sync_copy(rid_hbm.at[pl.ds(s * rpw + k * CHUNK, CHUNK)], sidx0)
        pltpu.sync_copy(rows0, acc_sh.at[sidx0])
    pltpu.sync_copy(ones_hbm, rows0)
    plsc.subcore_barrier()

    def cnt_chunk(base):
        pltpu.sync_copy(sidx_hbm.at[pl.ds(base, CHUNK)], sidx0)
        pltpu.sync_copy(oidx_hbm.at[pl.ds(base, CHUNK)], oidx0)
        pltpu.sync_copy(rows0, acc_sh.at[sidx0], add=True)
        pltpu.sync_copy(rows0, acc_sh.at[oidx0], add=True)

    def cbody(j, _):
        cnt_chunk((start + j) * CHUNK)
        return 0

    lax.fori_loop(0, CPW0, cbody, 0)

    @pl.when(w < NCHUNKS - NW * CPW0)
    def _():
        cnt_chunk((NW * CPW0 + w) * CHUNK)

    plsc.subcore_barrier()

    for k in range(rpw // CHUNK):
        off = s * rpw + k * CHUNK
        pltpu.sync_copy(rid_hbm.at[pl.ds(off, CHUNK)], sidx0)
        pltpu.async_copy(acc_sh.at[sidx0], rows0, sem1).wait()
        pltpu.sync_copy(rows0, counts_hbm.at[c, pl.ds(off, CHUNK)])


_sc_scatter = pl.kernel(
    _scatter_body,
    out_type=(jax.ShapeDtypeStruct((NC, OP, H), jnp.float32),
              jax.ShapeDtypeStruct((NC, OP, H), jnp.float32)),
    mesh=plsc.VectorSubcoreMesh(core_axis_name="c", subcore_axis_name="s"),
    scratch_types=[
        pltpu.VMEM_SHARED((OP, H), jnp.float32),
        pltpu.VMEM((CHUNK,), jnp.int32),
        pltpu.VMEM((CHUNK,), jnp.int32),
        pltpu.VMEM((CHUNK,), jnp.int32),
        pltpu.VMEM((CHUNK,), jnp.int32),
        pltpu.VMEM((CHUNK, H), jnp.float32),
        pltpu.VMEM((CHUNK, H), jnp.float32),
        pltpu.SemaphoreType.DMA,
        pltpu.SemaphoreType.DMA,
    ],
)


def _mlp2_body(p_ref, c_ref, w2a_ref, b2a_ref, w2b_ref, b2b_ref, out_ref):
    pooled = p_ref[0] + p_ref[1]
    cnt = jnp.maximum(c_ref[0, :, :1] + c_ref[1, :, :1], 1.0)
    pooled = pooled / cnt
    h = jnp.dot(pooled, w2a_ref[...], preferred_element_type=jnp.float32)
    h = jnp.maximum(h + b2a_ref[...], 0.0)
    out = jnp.dot(h, w2b_ref[...], preferred_element_type=jnp.float32)
    out_ref[...] = jnp.maximum(out + b2b_ref[...], 0.0)


def _tc_mlp2(pooled, counts, W2a, b2a, W2b, b2b):
    TM = 1024
    grid = (OP // TM,)
    return pl.pallas_call(
        _mlp2_body,
        grid=grid,
        in_specs=[
            pl.BlockSpec((NC, TM, H), lambda i: (0, i, 0)),
            pl.BlockSpec((NC, TM, H), lambda i: (0, i, 0)),
            pl.BlockSpec((H, H), lambda i: (0, 0)),
            pl.BlockSpec((1, H), lambda i: (0, 0)),
            pl.BlockSpec((H, DOUT), lambda i: (0, 0)),
            pl.BlockSpec((1, DOUT), lambda i: (0, 0)),
        ],
        out_specs=pl.BlockSpec((TM, DOUT), lambda i: (i, 0)),
        out_shape=jax.ShapeDtypeStruct((OP, DOUT), jnp.float32),
    )(pooled, counts, W2a, b2a.reshape(1, H), W2b, b2b.reshape(1, DOUT))


def kernel(obj_vecs, pred_vecs, edges, W1a, b1a, W1b, b1b, W2a, b2a, W2b, b2b):
    s_idx = edges[:, 0]
    o_idx = edges[:, 1]
    sgath, ogath = _sc_gather(obj_vecs, s_idx, o_idx)
    new_s, new_p, new_o = _tc_mlp1(sgath, pred_vecs, ogath, W1a, b1a, W1b, b1b)
    zeros = jnp.zeros((CHUNK, H), jnp.float32)
    ones = jnp.ones((CHUNK, H), jnp.float32)
    rid = jnp.arange(OP, dtype=jnp.int32)
    pooled, counts = _sc_scatter(new_s, new_o, s_idx, o_idx, zeros, ones, rid)
    new_obj_vecs = _tc_mlp2(pooled, counts, W2a, b2a, W2b, b2b)
    return (new_obj_vecs[:O], new_p)
